# baseline (speedup 1.0000x reference)
import functools

import jax
import jax.numpy as jnp
from jax import lax
from jax.experimental import pallas as pl
from jax.experimental.pallas import tpu as pltpu
from jax.experimental.pallas import tpu_sc as plsc

_BATCH = 16384
_EMB = 16
_LANE = 128


@functools.lru_cache(maxsize=None)
def _build_mf_kernel():
    info = plsc.get_sparse_core_info()
    nc, ns, nl = info.num_cores, info.num_subcores, info.num_lanes
    nw = nc * ns
    bpw = _BATCH // nw
    nchunks = bpw // nl
    pitch = nl + 1
    mesh = plsc.VectorSubcoreMesh(core_axis_name="c", subcore_axis_name="s")

    @functools.partial(
        pl.kernel,
        mesh=mesh,
        out_type=jax.ShapeDtypeStruct((_BATCH,), jnp.float32),
        compiler_params=pltpu.CompilerParams(needs_layout_passes=False),
        scratch_types=[
            pltpu.VMEM((bpw,), jnp.int32),
            pltpu.VMEM((bpw,), jnp.int32),
            pltpu.VMEM((nl, _EMB, _LANE), jnp.float32),
            pltpu.VMEM((nl, _EMB, _LANE), jnp.float32),
            pltpu.VMEM((nl, 1, _LANE), jnp.float32),
            pltpu.VMEM((nl, 1, _LANE), jnp.float32),
            pltpu.VMEM((nl,), jnp.float32),
            pltpu.VMEM((_EMB * (nl + 1),), jnp.float32),
            pltpu.VMEM((bpw,), jnp.float32),
            pltpu.VMEM((8704,), jnp.int32),
            pltpu.SemaphoreType.DMA,
        ],
    )
    def mf(u_id, i_id, uembT, ubiasT, iembT, ibiasT, mean16, out,
           uidx_v, iidx_v, ublk, iblk, ubb, ibb, mean_v, pt_v, out_v,
           drain_v, sem):
        wid = lax.axis_index("s") * nc + lax.axis_index("c")
        base = wid * bpw
        pltpu.sync_copy(u_id.at[pl.ds(base, bpw)], uidx_v)
        pltpu.sync_copy(i_id.at[pl.ds(base, bpw)], iidx_v)
        pltpu.sync_copy(mean16, mean_v)

        lanes = lax.iota(jnp.int32, nl)
        col = lanes * pitch
        mean_vec = mean_v[...]

        def chunk(c, carry):
            uv = uidx_v[pl.ds(c * nl, nl)]
            iv = iidx_v[pl.ds(c * nl, nl)]
            ucol = uv & (_LANE - 1)
            icol = iv & (_LANE - 1)
            ualn = uv - ucol
            ialn = iv - icol
            for l in range(nl):
                ua = pl.multiple_of(ualn[l], _LANE)
                ia = pl.multiple_of(ialn[l], _LANE)
                pltpu.async_copy(uembT.at[:, pl.ds(ua, _LANE)], ublk.at[l],
                                 sem)
                pltpu.async_copy(iembT.at[:, pl.ds(ia, _LANE)], iblk.at[l],
                                 sem)
                pltpu.async_copy(ubiasT.at[:, pl.ds(ua, _LANE)], ubb.at[l],
                                 sem)
                pltpu.async_copy(ibiasT.at[:, pl.ds(ia, _LANE)], ibb.at[l],
                                 sem)
            for _ in range(8):
                pltpu.make_async_copy(
                    u_id.at[pl.ds(0, 8704)], drain_v, sem).wait()
            zer = jnp.zeros((nl,), jnp.int32)
            ubv = plsc.load_gather(ubb, [lanes, zer, ucol])
            ibv = plsc.load_gather(ibb, [lanes, zer, icol])
            acc = ubv + ibv + mean_vec
            for l in range(nl):
                lv = jnp.full((nl,), l, jnp.int32)
                uc = plsc.load_gather(ublk, [lv, lanes, zer + ucol[l]])
                ic = plsc.load_gather(iblk, [lv, lanes, zer + icol[l]])
                plsc.store_scatter(pt_v, [col + l], uc * ic)
            for d in range(_EMB):
                acc = acc + pt_v[pl.ds(d * pitch, nl)]
            out_v[pl.ds(c * nl, nl)] = 1.0 / (1.0 + jnp.exp(-acc))
            return carry

        lax.fori_loop(0, nchunks, chunk, 0)
        pltpu.sync_copy(out_v, out.at[pl.ds(base, bpw)])

    return mf


def kernel(data, user_emb, user_bias, item_emb, item_bias, mean):
    u_id = data[0].astype(jnp.int32)
    i_id = data[1].astype(jnp.int32)
    mean16 = jnp.broadcast_to(mean.astype(jnp.float32), (16,))
    mf = _build_mf_kernel()
    return mf(u_id, i_id, user_emb.T, user_bias.T, item_emb.T, item_bias.T,
              mean16)


# speedup vs baseline: 1.0653x; 1.0653x over previous
import functools

import jax
import jax.numpy as jnp
from jax import lax
from jax.experimental import pallas as pl
from jax.experimental.pallas import tpu as pltpu
from jax.experimental.pallas import tpu_sc as plsc

_BATCH = 16384
_EMB = 16
_LANE = 128

@functools.lru_cache(maxsize=None)
def _build_mf_kernel():
    info = plsc.get_sparse_core_info()
    nc, ns, nl = info.num_cores, info.num_subcores, info.num_lanes
    nw = nc * ns
    bpw = _BATCH // nw
    nchunks = bpw // nl
    mesh = plsc.VectorSubcoreMesh(core_axis_name="c", subcore_axis_name="s")

    @functools.partial(
        pl.kernel,
        mesh=mesh,
        out_type=jax.ShapeDtypeStruct((_BATCH,), jnp.float32),
        compiler_params=pltpu.CompilerParams(needs_layout_passes=False),
        scratch_types=[
            pltpu.VMEM((bpw,), jnp.int32),
            pltpu.VMEM((bpw,), jnp.int32),
            pltpu.VMEM((nl, _EMB, _LANE), jnp.float32),
            pltpu.VMEM((nl, _EMB, _LANE), jnp.float32),
            pltpu.VMEM((nl, 1, _LANE), jnp.float32),
            pltpu.VMEM((nl, 1, _LANE), jnp.float32),
            pltpu.VMEM((nl,), jnp.float32),
            pltpu.VMEM((bpw,), jnp.float32),
            pltpu.SemaphoreType.DMA,
        ],
    )
    def mf(u_id, i_id, uembT, ubiasT, iembT, ibiasT, mean16, out,
           uidx_v, iidx_v, ublk, iblk, ubb, ibb, mean_v, out_v, sem):
        wid = lax.axis_index("s") * nc + lax.axis_index("c")
        base = wid * bpw
        pltpu.sync_copy(u_id.at[pl.ds(base, bpw)], uidx_v)
        pltpu.sync_copy(i_id.at[pl.ds(base, bpw)], iidx_v)
        pltpu.sync_copy(mean16, mean_v)

        lanes = lax.iota(jnp.int32, nl)
        mean_vec = mean_v[...]

        def chunk(c, carry):
            uv = uidx_v[pl.ds(c * nl, nl)]
            iv = iidx_v[pl.ds(c * nl, nl)]
            ucol = uv & (_LANE - 1)
            icol = iv & (_LANE - 1)
            ualn = uv - ucol
            ialn = iv - icol
            copies = []
            for l in range(nl):
                ua = pl.multiple_of(ualn[l], _LANE)
                ia = pl.multiple_of(ialn[l], _LANE)
                copies.append(pltpu.async_copy(
                    uembT.at[:, pl.ds(ua, _LANE)],
                    ublk.at[l], sem))
                copies.append(pltpu.async_copy(
                    iembT.at[:, pl.ds(ia, _LANE)],
                    iblk.at[l], sem))
                copies.append(pltpu.async_copy(
                    ubiasT.at[:, pl.ds(ua, _LANE)],
                    ubb.at[l], sem))
                copies.append(pltpu.async_copy(
                    ibiasT.at[:, pl.ds(ia, _LANE)],
                    ibb.at[l], sem))
            for cp in copies:
                cp.wait()
            zer = jnp.zeros((nl,), jnp.int32)
            ubv = plsc.load_gather(ubb, [lanes, zer, ucol])
            ibv = plsc.load_gather(ibb, [lanes, zer, icol])
            acc = ubv + ibv + mean_vec
            for d in range(_EMB):
                dv = jnp.full((nl,), d, jnp.int32)
                uvals = plsc.load_gather(ublk, [lanes, dv, ucol])
                ivals = plsc.load_gather(iblk, [lanes, dv, icol])
                acc = acc + uvals * ivals
            out_v[pl.ds(c * nl, nl)] = 1.0 / (1.0 + jnp.exp(-acc))
            return carry

        lax.fori_loop(0, nchunks, chunk, 0)
        pltpu.sync_copy(out_v, out.at[pl.ds(base, bpw)])

    return mf


def kernel(data, user_emb, user_bias, item_emb, item_bias, mean):
    u_id = data[0].astype(jnp.int32)
    i_id = data[1].astype(jnp.int32)
    mean16 = jnp.broadcast_to(mean.astype(jnp.float32), (16,))
    mf = _build_mf_kernel()
    return mf(u_id, i_id, user_emb.T, user_bias.T, item_emb.T, item_bias.T,
              mean16)


# speedup vs baseline: 1.0694x; 1.0039x over previous
import functools

import jax
import jax.numpy as jnp
from jax import lax
from jax.experimental import pallas as pl
from jax.experimental.pallas import tpu as pltpu
from jax.experimental.pallas import tpu_sc as plsc

_BATCH = 16384
_EMB = 16
_LANE = 128
_H = 8
_HSET_I32 = _H * (2 * _EMB * _LANE + 2 * _LANE)
_DRN = _HSET_I32 // 4


@functools.lru_cache(maxsize=None)
def _build_mf_kernel():
    info = plsc.get_sparse_core_info()
    nc, ns, nl = info.num_cores, info.num_subcores, info.num_lanes
    nw = nc * ns
    bpw = _BATCH // nw
    nchunks = bpw // nl
    mesh = plsc.VectorSubcoreMesh(core_axis_name="c", subcore_axis_name="s")

    @functools.partial(
        pl.kernel,
        mesh=mesh,
        out_type=jax.ShapeDtypeStruct((_BATCH,), jnp.float32),
        compiler_params=pltpu.CompilerParams(needs_layout_passes=False),
        scratch_types=[
            pltpu.VMEM((bpw,), jnp.int32),
            pltpu.VMEM((bpw,), jnp.int32),
            pltpu.VMEM((_H, _EMB, _LANE), jnp.float32),
            pltpu.VMEM((_H, _EMB, _LANE), jnp.float32),
            pltpu.VMEM((_H, 1, _LANE), jnp.float32),
            pltpu.VMEM((_H, 1, _LANE), jnp.float32),
            pltpu.VMEM((_H, _EMB, _LANE), jnp.float32),
            pltpu.VMEM((_H, _EMB, _LANE), jnp.float32),
            pltpu.VMEM((_H, 1, _LANE), jnp.float32),
            pltpu.VMEM((_H, 1, _LANE), jnp.float32),
            pltpu.VMEM((nl,), jnp.float32),
            pltpu.VMEM((bpw,), jnp.float32),
            pltpu.VMEM((_DRN,), jnp.int32),
            pltpu.SemaphoreType.DMA,
            pltpu.SemaphoreType.DMA,
        ],
    )
    def mf(u_id, i_id, uembT, ubiasT, iembT, ibiasT, mean16, out,
           uidx_v, iidx_v, ue_a, ie_a, ub_a, ib_a, ue_b, ie_b, ub_b, ib_b,
           mean_v, out_v, drain_v, sem_a, sem_b):
        wid = lax.axis_index("s") * nc + lax.axis_index("c")
        base = wid * bpw
        pltpu.sync_copy(u_id.at[pl.ds(base, bpw)], uidx_v)
        pltpu.sync_copy(i_id.at[pl.ds(base, bpw)], iidx_v)
        pltpu.sync_copy(mean16, mean_v)

        lanes = lax.iota(jnp.int32, nl)
        mean_vec = mean_v[...]

        def load_ids(c):
            return (uidx_v[pl.ds(c * nl, nl)], iidx_v[pl.ds(c * nl, nl)])

        def issue(uv, iv, half, ue, ie, ub, ib, sem):
            ualn = uv - (uv & (_LANE - 1))
            ialn = iv - (iv & (_LANE - 1))
            for j in range(_H):
                l = half * _H + j
                ua = pl.multiple_of(ualn[l], _LANE)
                ia = pl.multiple_of(ialn[l], _LANE)
                pltpu.async_copy(uembT.at[:, pl.ds(ua, _LANE)], ue.at[j], sem)
                pltpu.async_copy(iembT.at[:, pl.ds(ia, _LANE)], ie.at[j], sem)
                pltpu.async_copy(ubiasT.at[:, pl.ds(ua, _LANE)], ub.at[j],
                                 sem)
                pltpu.async_copy(ibiasT.at[:, pl.ds(ia, _LANE)], ib.at[j],
                                 sem)

        def drain(sem):
            for _ in range(4):
                pltpu.make_async_copy(
                    u_id.at[pl.ds(0, _DRN)], drain_v, sem).wait()

        def compute(uv, iv, half, ue, ie, ub, ib):
            slot = jnp.clip(lanes - half * _H, 0, _H - 1)
            ucol = uv & (_LANE - 1)
            icol = iv & (_LANE - 1)
            zer = jnp.zeros((nl,), jnp.int32)
            acc = (plsc.load_gather(ub, [slot, zer, ucol])
                   + plsc.load_gather(ib, [slot, zer, icol]) + mean_vec)
            for d in range(_EMB):
                dv = jnp.full((nl,), d, jnp.int32)
                acc = acc + (plsc.load_gather(ue, [slot, dv, ucol])
                             * plsc.load_gather(ie, [slot, dv, icol]))
            return acc

        uv0, iv0 = load_ids(0)
        issue(uv0, iv0, 0, ue_a, ie_a, ub_a, ib_a, sem_a)

        def chunk(c, carry):
            uv, iv = load_ids(c)
            issue(uv, iv, 1, ue_b, ie_b, ub_b, ib_b, sem_b)
            drain(sem_a)
            acc_a = compute(uv, iv, 0, ue_a, ie_a, ub_a, ib_a)

            @pl.when(c < nchunks - 1)
            def _():
                uv2, iv2 = load_ids(c + 1)
                issue(uv2, iv2, 0, ue_a, ie_a, ub_a, ib_a, sem_a)

            drain(sem_b)
            acc_b = compute(uv, iv, 1, ue_b, ie_b, ub_b, ib_b)
            acc = jnp.where(lanes < _H, acc_a, acc_b)
            out_v[pl.ds(c * nl, nl)] = 1.0 / (1.0 + jnp.exp(-acc))
            return carry

        lax.fori_loop(0, nchunks, chunk, 0)
        pltpu.sync_copy(out_v, out.at[pl.ds(base, bpw)])

    return mf


def kernel(data, user_emb, user_bias, item_emb, item_bias, mean):
    u_id = data[0].astype(jnp.int32)
    i_id = data[1].astype(jnp.int32)
    mean16 = jnp.broadcast_to(mean.astype(jnp.float32), (16,))
    mf = _build_mf_kernel()
    return mf(u_id, i_id, user_emb.T, user_bias.T, item_emb.T, item_bias.T,
              mean16)
